# baseline (speedup 1.0000x reference)
import jax
import jax.numpy as jnp
from jax.experimental import pallas as pl
from jax.experimental.pallas import tpu as pltpu

_S = 16
_TB = 128
_TB2 = 2048


def _gcn_body(feat_ref, adjt_ref, w_ref, bias_ref, a_ref, node_ref, ctx_ref):
    S = _S
    TB = _TB
    w = w_ref[...]
    hid, in_dim = w.shape
    a = a_ref[0]
    bias = jnp.broadcast_to(bias_ref[...], (hid, TB))
    dn = (((1,), (1,)), ((), ()))

    m = []
    for t in range(S):
        ft = feat_ref[:, t * in_dim:(t + 1) * in_dim]
        m.append(jax.lax.dot_general(
            w, ft, dn, preferred_element_type=jnp.float32))

    ctx_acc = jnp.zeros((hid, TB), jnp.float32)
    for s in range(S):
        acc = bias
        for t in range(S):
            row = adjt_ref[s * S + t:s * S + t + 1, :]
            acc = acc + jnp.broadcast_to(row, (hid, TB)) * m[t]
        h = jnp.where(acc >= 0, acc, a * acc)
        if s < S - 1:
            ctx_acc = ctx_acc + h
        else:
            node_ref[...] = h
    ctx_ref[...] = ctx_acc * (1.0 / (S - 1))


def _bil_body(node_ref, ctx_ref, ctxs_ref, w_ref, b_ref, pos_ref, neg_ref):
    t = jnp.dot(w_ref[...].T, node_ref[...],
                preferred_element_type=jnp.float32)
    b = b_ref[0]
    pos_ref[...] = jnp.sum(t * ctx_ref[...], axis=0, keepdims=True) + b
    neg_ref[...] = jnp.sum(t * ctxs_ref[...], axis=0, keepdims=True) + b


def kernel(feature_seq, adj_matrix, W_gcn, gcn_bias, prelu_a, bilinear_W,
           bilinear_b):
    B, S, IN = feature_seq.shape
    HID = W_gcn.shape[0]
    adjt = adj_matrix.transpose(1, 2, 0).reshape(S * S, B)
    feat2d = feature_seq.reshape(B, S * IN)
    bias2 = gcn_bias.reshape(HID, 1)
    a1 = prelu_a.reshape(1)

    node_t, ctx_t = pl.pallas_call(
        _gcn_body,
        grid=(B // _TB,),
        in_specs=[
            pl.BlockSpec((_TB, S * IN), lambda i: (i, 0)),
            pl.BlockSpec((S * S, _TB), lambda i: (0, i)),
            pl.BlockSpec((HID, IN), lambda i: (0, 0)),
            pl.BlockSpec((HID, 1), lambda i: (0, 0)),
            pl.BlockSpec(memory_space=pltpu.SMEM),
        ],
        out_specs=[
            pl.BlockSpec((HID, _TB), lambda i: (0, i)),
            pl.BlockSpec((HID, _TB), lambda i: (0, i)),
        ],
        out_shape=[
            jax.ShapeDtypeStruct((HID, B), jnp.float32),
            jax.ShapeDtypeStruct((HID, B), jnp.float32),
        ],
    )(feat2d, adjt, W_gcn, bias2, a1)

    ctxs_t = jnp.concatenate([ctx_t[:, B - 2:B - 1], ctx_t[:, :B - 1]], axis=1)
    wb = bilinear_W.reshape(HID, HID)
    bb = bilinear_b.reshape(1)

    pos_t, neg_t = pl.pallas_call(
        _bil_body,
        grid=(B // _TB2,),
        in_specs=[
            pl.BlockSpec((HID, _TB2), lambda i: (0, i)),
            pl.BlockSpec((HID, _TB2), lambda i: (0, i)),
            pl.BlockSpec((HID, _TB2), lambda i: (0, i)),
            pl.BlockSpec((HID, HID), lambda i: (0, 0)),
            pl.BlockSpec(memory_space=pltpu.SMEM),
        ],
        out_specs=[
            pl.BlockSpec((1, _TB2), lambda i: (0, i)),
            pl.BlockSpec((1, _TB2), lambda i: (0, i)),
        ],
        out_shape=[
            jax.ShapeDtypeStruct((1, B), jnp.float32),
            jax.ShapeDtypeStruct((1, B), jnp.float32),
        ],
    )(node_t, ctx_t, ctxs_t, wb, bb)

    logits = jnp.concatenate([pos_t, neg_t], axis=1).reshape(2 * B, 1)
    return (logits, node_t.T, ctx_t.T)


# speedup vs baseline: 1.7381x; 1.7381x over previous
import jax
import jax.numpy as jnp
from jax.experimental import pallas as pl
from jax.experimental.pallas import tpu as pltpu

_S = 16
_TB = 128
_TB2 = 2048


def _gcn_body(feat_ref, adjt_ref, w_ref, bias_ref, a_ref, node_ref, ctx_ref):
    S = _S
    TB = _TB
    w = w_ref[...]
    hid, in_dim = w.shape
    a = a_ref[0]
    bias = jnp.broadcast_to(bias_ref[...], (hid, TB))
    dn = (((1,), (1,)), ((), ()))

    m = []
    for t in range(S):
        ft = feat_ref[:, t, :]
        m.append(jax.lax.dot_general(
            w, ft, dn, preferred_element_type=jnp.float32))

    ctx_acc = jnp.zeros((hid, TB), jnp.float32)
    for s in range(S):
        acc = bias
        for t in range(S):
            row = adjt_ref[s * S + t:s * S + t + 1, :]
            acc = acc + jnp.broadcast_to(row, (hid, TB)) * m[t]
        h = jnp.where(acc >= 0, acc, a * acc)
        if s < S - 1:
            ctx_acc = ctx_acc + h
        else:
            node_ref[...] = h
    ctx_ref[...] = ctx_acc * (1.0 / (S - 1))


def _bil_body(node_ref, ctx_ref, ctxs_ref, w_ref, b_ref, pos_ref, neg_ref):
    t = jnp.dot(w_ref[...].T, node_ref[...],
                preferred_element_type=jnp.float32)
    b = b_ref[0]
    pos_ref[...] = jnp.sum(t * ctx_ref[...], axis=0, keepdims=True) + b
    neg_ref[...] = jnp.sum(t * ctxs_ref[...], axis=0, keepdims=True) + b


def kernel(feature_seq, adj_matrix, W_gcn, gcn_bias, prelu_a, bilinear_W,
           bilinear_b):
    B, S, IN = feature_seq.shape
    HID = W_gcn.shape[0]
    adjt = adj_matrix.transpose(1, 2, 0).reshape(S * S, B)
    bias2 = gcn_bias.reshape(HID, 1)
    a1 = prelu_a.reshape(1)

    node_t, ctx_t = pl.pallas_call(
        _gcn_body,
        grid=(B // _TB,),
        in_specs=[
            pl.BlockSpec((_TB, S, IN), lambda i: (i, 0, 0)),
            pl.BlockSpec((S * S, _TB), lambda i: (0, i)),
            pl.BlockSpec((HID, IN), lambda i: (0, 0)),
            pl.BlockSpec((HID, 1), lambda i: (0, 0)),
            pl.BlockSpec(memory_space=pltpu.SMEM),
        ],
        out_specs=[
            pl.BlockSpec((HID, _TB), lambda i: (0, i)),
            pl.BlockSpec((HID, _TB), lambda i: (0, i)),
        ],
        out_shape=[
            jax.ShapeDtypeStruct((HID, B), jnp.float32),
            jax.ShapeDtypeStruct((HID, B), jnp.float32),
        ],
    )(feature_seq, adjt, W_gcn, bias2, a1)

    ctxs_t = jnp.concatenate([ctx_t[:, B - 2:B - 1], ctx_t[:, :B - 1]], axis=1)
    wb = bilinear_W.reshape(HID, HID)
    bb = bilinear_b.reshape(1)

    pos_t, neg_t = pl.pallas_call(
        _bil_body,
        grid=(B // _TB2,),
        in_specs=[
            pl.BlockSpec((HID, _TB2), lambda i: (0, i)),
            pl.BlockSpec((HID, _TB2), lambda i: (0, i)),
            pl.BlockSpec((HID, _TB2), lambda i: (0, i)),
            pl.BlockSpec((HID, HID), lambda i: (0, 0)),
            pl.BlockSpec(memory_space=pltpu.SMEM),
        ],
        out_specs=[
            pl.BlockSpec((1, _TB2), lambda i: (0, i)),
            pl.BlockSpec((1, _TB2), lambda i: (0, i)),
        ],
        out_shape=[
            jax.ShapeDtypeStruct((1, B), jnp.float32),
            jax.ShapeDtypeStruct((1, B), jnp.float32),
        ],
    )(node_t, ctx_t, ctxs_t, wb, bb)

    logits = jnp.concatenate([pos_t, neg_t], axis=1).reshape(2 * B, 1)
    return (logits, node_t.T, ctx_t.T)


# speedup vs baseline: 1.7469x; 1.0050x over previous
import jax
import jax.numpy as jnp
from jax.experimental import pallas as pl
from jax.experimental.pallas import tpu as pltpu

_S = 16
_TB = 256
_TB2 = 2048


def _gcn_body(feat_ref, adjt_ref, w_ref, bias_ref, a_ref, node_ref, ctx_ref):
    S = _S
    TB = _TB
    w = w_ref[...]
    hid, in_dim = w.shape
    a = a_ref[0]
    bias = jnp.broadcast_to(bias_ref[...], (hid, TB))
    dn = (((1,), (1,)), ((), ()))

    m = []
    for t in range(S):
        ft = feat_ref[:, t, :]
        m.append(jax.lax.dot_general(
            w, ft, dn, preferred_element_type=jnp.float32))

    ctx_acc = jnp.zeros((hid, TB), jnp.float32)
    for s in range(S):
        acc = bias
        for t in range(S):
            row = adjt_ref[s * S + t:s * S + t + 1, :]
            acc = acc + jnp.broadcast_to(row, (hid, TB)) * m[t]
        h = jnp.where(acc >= 0, acc, a * acc)
        if s < S - 1:
            ctx_acc = ctx_acc + h
        else:
            node_ref[...] = h
    ctx_ref[...] = ctx_acc * (1.0 / (S - 1))


def _bil_body(node_ref, ctx_ref, ctxs_ref, w_ref, b_ref, pos_ref, neg_ref):
    t = jnp.dot(w_ref[...].T, node_ref[...],
                preferred_element_type=jnp.float32)
    b = b_ref[0]
    pos_ref[...] = jnp.sum(t * ctx_ref[...], axis=0, keepdims=True) + b
    neg_ref[...] = jnp.sum(t * ctxs_ref[...], axis=0, keepdims=True) + b


def kernel(feature_seq, adj_matrix, W_gcn, gcn_bias, prelu_a, bilinear_W,
           bilinear_b):
    B, S, IN = feature_seq.shape
    HID = W_gcn.shape[0]
    adjt = adj_matrix.transpose(1, 2, 0).reshape(S * S, B)
    bias2 = gcn_bias.reshape(HID, 1)
    a1 = prelu_a.reshape(1)

    node_t, ctx_t = pl.pallas_call(
        _gcn_body,
        grid=(B // _TB,),
        in_specs=[
            pl.BlockSpec((_TB, S, IN), lambda i: (i, 0, 0)),
            pl.BlockSpec((S * S, _TB), lambda i: (0, i)),
            pl.BlockSpec((HID, IN), lambda i: (0, 0)),
            pl.BlockSpec((HID, 1), lambda i: (0, 0)),
            pl.BlockSpec(memory_space=pltpu.SMEM),
        ],
        out_specs=[
            pl.BlockSpec((HID, _TB), lambda i: (0, i)),
            pl.BlockSpec((HID, _TB), lambda i: (0, i)),
        ],
        out_shape=[
            jax.ShapeDtypeStruct((HID, B), jnp.float32),
            jax.ShapeDtypeStruct((HID, B), jnp.float32),
        ],
    )(feature_seq, adjt, W_gcn, bias2, a1)

    ctxs_t = jnp.concatenate([ctx_t[:, B - 2:B - 1], ctx_t[:, :B - 1]], axis=1)
    wb = bilinear_W.reshape(HID, HID)
    bb = bilinear_b.reshape(1)

    pos_t, neg_t = pl.pallas_call(
        _bil_body,
        grid=(B // _TB2,),
        in_specs=[
            pl.BlockSpec((HID, _TB2), lambda i: (0, i)),
            pl.BlockSpec((HID, _TB2), lambda i: (0, i)),
            pl.BlockSpec((HID, _TB2), lambda i: (0, i)),
            pl.BlockSpec((HID, HID), lambda i: (0, 0)),
            pl.BlockSpec(memory_space=pltpu.SMEM),
        ],
        out_specs=[
            pl.BlockSpec((1, _TB2), lambda i: (0, i)),
            pl.BlockSpec((1, _TB2), lambda i: (0, i)),
        ],
        out_shape=[
            jax.ShapeDtypeStruct((1, B), jnp.float32),
            jax.ShapeDtypeStruct((1, B), jnp.float32),
        ],
    )(node_t, ctx_t, ctxs_t, wb, bb)

    logits = jnp.concatenate([pos_t, neg_t], axis=1).reshape(2 * B, 1)
    return (logits, node_t.T, ctx_t.T)
